# Initial kernel scaffold; baseline (speedup 1.0000x reference)
import functools

import jax
import jax.numpy as jnp
from jax import lax
from jax.experimental import pallas as pl
from jax.experimental.pallas import tpu as pltpu
from jax.experimental.pallas import tpu_sc as plsc

N = 10000
E = 320000
DF = 128
DH = 256
DO = 128

NC = 2
NS = 16
NW = NC * NS
EPT = E // NW
CH = 80
NCHUNK = EPT // CH
ZROWS = 80

_mesh = plsc.VectorSubcoreMesh(
    core_axis_name="c", subcore_axis_name="s", num_cores=NC, num_subcores=NS
)


@functools.partial(
    pl.kernel,
    out_type=jax.ShapeDtypeStruct((NC * N, DF), jnp.float32),
    mesh=_mesh,
    scratch_types=[
        pltpu.VMEM((EPT,), jnp.int32),
        pltpu.VMEM((EPT,), jnp.int32),
        pltpu.VMEM((EPT,), jnp.float32),
        pltpu.VMEM((CH, DF), jnp.float32),
        pltpu.VMEM((CH,), jnp.int32),
        pltpu.VMEM_SHARED((N, DF), jnp.float32),
    ],
)
def _agg(src_hbm, dst_hbm, w_hbm, feat_hbm, out_hbm,
         src_v, dst_v, w_v, rows_v, dstb_v, acc_sh):
    c = lax.axis_index("c")
    s = lax.axis_index("s")
    tile = c * NS + s
    ebase = tile * EPT

    pltpu.sync_copy(src_hbm.at[pl.ds(ebase, EPT)], src_v)
    pltpu.sync_copy(dst_hbm.at[pl.ds(ebase, EPT)], dst_v)
    pltpu.sync_copy(w_hbm.at[pl.ds(ebase, EPT)], w_v)

    zero16 = jnp.zeros((16,), jnp.float32)

    def _zrow(e, _):
        for k in range(DF // 16):
            rows_v[e, pl.ds(k * 16, 16)] = zero16
        return 0

    lax.fori_loop(0, CH, _zrow, 0)

    @pl.when(s < NS - 1)
    def _():
        for i in range(640 // ZROWS):
            pltpu.sync_copy(rows_v, acc_sh.at[pl.ds(s * 640 + i * ZROWS, ZROWS)])

    @pl.when(s == NS - 1)
    def _():
        for i in range(400 // ZROWS):
            pltpu.sync_copy(rows_v, acc_sh.at[pl.ds(9600 + i * ZROWS, ZROWS)])

    plsc.subcore_barrier()

    def _chunk(j, _):
        eoff = j * CH
        pltpu.sync_copy(feat_hbm.at[src_v.at[pl.ds(eoff, CH)]], rows_v)

        def _edge(e, _):
            wv = w_v[eoff + e]
            for k in range(DF // 16):
                sl = pl.ds(k * 16, 16)
                rows_v[e, sl] = rows_v[e, sl] * wv
            return 0

        lax.fori_loop(0, CH, _edge, 0)

        pltpu.sync_copy(dst_v.at[pl.ds(eoff, CH)], dstb_v)
        pltpu.sync_copy(rows_v, acc_sh.at[dstb_v], add=True)
        return 0

    lax.fori_loop(0, NCHUNK, _chunk, 0)

    plsc.subcore_barrier()

    obase = c * N

    @pl.when(s < NS - 1)
    def _():
        for i in range(640 // ZROWS):
            r0 = s * 640 + i * ZROWS
            pltpu.sync_copy(acc_sh.at[pl.ds(r0, ZROWS)], rows_v)
            pltpu.sync_copy(rows_v, out_hbm.at[pl.ds(obase + r0, ZROWS)])

    @pl.when(s == NS - 1)
    def _():
        for i in range(400 // ZROWS):
            r0 = 9600 + i * ZROWS
            pltpu.sync_copy(acc_sh.at[pl.ds(r0, ZROWS)], rows_v)
            pltpu.sync_copy(rows_v, out_hbm.at[pl.ds(obase + r0, ZROWS)])


ROWS_BLK = 1000
GRID1 = N // ROWS_BLK


def _dense1_body(p0_ref, p1_ref, wg_ref, bg_ref, w1_ref, b1_ref,
                 y_ref, s1_ref, s2_ref):
    agg = p0_ref[...] + p1_ref[...]
    x1 = jnp.dot(agg, wg_ref[...], preferred_element_type=jnp.float32)
    x1 = jnp.maximum(x1 + bg_ref[...], 0.0)
    y = jnp.dot(x1, w1_ref[...], preferred_element_type=jnp.float32) + b1_ref[...]
    y_ref[...] = y

    @pl.when(pl.program_id(0) == 0)
    def _():
        s1_ref[...] = jnp.zeros_like(s1_ref)
        s2_ref[...] = jnp.zeros_like(s2_ref)

    s1_ref[...] += jnp.sum(y, axis=0, keepdims=True)
    s2_ref[...] += jnp.sum(y * y, axis=0, keepdims=True)


def _dense2_body(y_ref, s1_ref, s2_ref, gamma_ref, beta_ref, w2_ref, b2_ref,
                 out_ref):
    mean = s1_ref[...] / N
    var = s2_ref[...] / N - mean * mean
    inv = gamma_ref[...] * lax.rsqrt(var + 1e-5)
    xh = (y_ref[...] - mean) * inv + beta_ref[...]
    z = jnp.maximum(xh, 0.0)
    out_ref[...] = (
        jnp.dot(z, w2_ref[...], preferred_element_type=jnp.float32) + b2_ref[...]
    )


def kernel(edge_index, edge_weight, feat, W_gcn, b_gcn, W1, b1, gamma, beta,
           W2, b2):
    src = edge_index[0].astype(jnp.int32)
    dst = edge_index[1].astype(jnp.int32)

    partials = _agg(src, dst, edge_weight, feat)
    p0 = partials[:N]
    p1 = partials[N:]

    bg2 = b_gcn.reshape(1, DH)
    b12 = b1.reshape(1, DH)
    g2 = gamma.reshape(1, DH)
    be2 = beta.reshape(1, DH)
    b22 = b2.reshape(1, DO)

    y, s1, s2 = pl.pallas_call(
        _dense1_body,
        grid=(GRID1,),
        in_specs=[
            pl.BlockSpec((ROWS_BLK, DF), lambda i: (i, 0)),
            pl.BlockSpec((ROWS_BLK, DF), lambda i: (i, 0)),
            pl.BlockSpec((DF, DH), lambda i: (0, 0)),
            pl.BlockSpec((1, DH), lambda i: (0, 0)),
            pl.BlockSpec((DH, DH), lambda i: (0, 0)),
            pl.BlockSpec((1, DH), lambda i: (0, 0)),
        ],
        out_specs=[
            pl.BlockSpec((ROWS_BLK, DH), lambda i: (i, 0)),
            pl.BlockSpec((1, DH), lambda i: (0, 0)),
            pl.BlockSpec((1, DH), lambda i: (0, 0)),
        ],
        out_shape=[
            jax.ShapeDtypeStruct((N, DH), jnp.float32),
            jax.ShapeDtypeStruct((1, DH), jnp.float32),
            jax.ShapeDtypeStruct((1, DH), jnp.float32),
        ],
    )(p0, p1, W_gcn, bg2, W1, b12)

    out = pl.pallas_call(
        _dense2_body,
        grid=(GRID1,),
        in_specs=[
            pl.BlockSpec((ROWS_BLK, DH), lambda i: (i, 0)),
            pl.BlockSpec((1, DH), lambda i: (0, 0)),
            pl.BlockSpec((1, DH), lambda i: (0, 0)),
            pl.BlockSpec((1, DH), lambda i: (0, 0)),
            pl.BlockSpec((1, DH), lambda i: (0, 0)),
            pl.BlockSpec((DH, DO), lambda i: (0, 0)),
            pl.BlockSpec((1, DO), lambda i: (0, 0)),
        ],
        out_specs=pl.BlockSpec((ROWS_BLK, DO), lambda i: (i, 0)),
        out_shape=jax.ShapeDtypeStruct((N, DO), jnp.float32),
    )(y, s1, s2, g2, be2, W2, b22)

    return out


# speedup vs baseline: 8.3251x; 8.3251x over previous
import functools

import jax
import jax.numpy as jnp
from jax import lax
from jax.experimental import pallas as pl
from jax.experimental.pallas import tpu as pltpu
from jax.experimental.pallas import tpu_sc as plsc

N = 10000
E = 320000
DF = 128
DH = 256
DO = 128

NC = 2
NS = 16
NW = NC * NS
EPT = E // NW
CH = 80
NCHUNK = EPT // CH
ZROWS = 80

@functools.cache
def _build_agg():
    mesh = plsc.VectorSubcoreMesh(
        core_axis_name="c", subcore_axis_name="s", num_cores=NC, num_subcores=NS
    )
    return functools.partial(
        pl.kernel,
        out_type=jax.ShapeDtypeStruct((NC * N, DF), jnp.float32),
        mesh=mesh,
        scratch_types=[
            pltpu.VMEM((EPT,), jnp.int32),
            pltpu.VMEM((NCHUNK, CH), jnp.int32),
            pltpu.VMEM((EPT,), jnp.float32),
            pltpu.VMEM((CH, DF), jnp.float32),
            pltpu.VMEM_SHARED((N, DF), jnp.float32),
        ],
    )(_agg_body)


def _agg_body(src_hbm, dst_hbm, w_hbm, feat_hbm, out_hbm,
              src_v, dst_v, w_v, rows_v, acc_sh):
    c = lax.axis_index("c")
    s = lax.axis_index("s")
    tile = c * NS + s
    ebase = tile * EPT

    pltpu.sync_copy(src_hbm.at[pl.ds(ebase, EPT)], src_v)
    pltpu.sync_copy(dst_hbm.at[tile], dst_v)
    pltpu.sync_copy(w_hbm.at[pl.ds(ebase, EPT)], w_v)

    zero16 = jnp.zeros((16,), jnp.float32)

    def _zrow(e, _):
        for k in range(DF // 16):
            rows_v[e, pl.ds(k * 16, 16)] = zero16
        return 0

    lax.fori_loop(0, CH, _zrow, 0)

    @pl.when(s < NS - 1)
    def _():
        for i in range(640 // ZROWS):
            pltpu.sync_copy(rows_v, acc_sh.at[pl.ds(s * 640 + i * ZROWS, ZROWS)])

    @pl.when(s == NS - 1)
    def _():
        for i in range(400 // ZROWS):
            pltpu.sync_copy(rows_v, acc_sh.at[pl.ds(9600 + i * ZROWS, ZROWS)])

    plsc.subcore_barrier()

    def _chunk(j, _):
        eoff = j * CH
        pltpu.sync_copy(feat_hbm.at[src_v.at[pl.ds(eoff, CH)]], rows_v)

        def _group(g, _):
            wvec = w_v[pl.ds(eoff + g * 16, 16)]
            for e16 in range(16):
                wv = wvec[e16]
                ridx = g * 16 + e16
                for k in range(DF // 16):
                    sl = pl.ds(k * 16, 16)
                    rows_v[ridx, sl] = rows_v[ridx, sl] * wv
            return 0

        lax.fori_loop(0, CH // 16, _group, 0)

        pltpu.sync_copy(rows_v, acc_sh.at[dst_v.at[j]], add=True)
        return 0

    lax.fori_loop(0, NCHUNK, _chunk, 0)

    plsc.subcore_barrier()

    obase = c * N

    @pl.when(s < NS - 1)
    def _():
        for i in range(640 // ZROWS):
            r0 = s * 640 + i * ZROWS
            pltpu.sync_copy(acc_sh.at[pl.ds(r0, ZROWS)], rows_v)
            pltpu.sync_copy(rows_v, out_hbm.at[pl.ds(obase + r0, ZROWS)])

    @pl.when(s == NS - 1)
    def _():
        for i in range(400 // ZROWS):
            r0 = 9600 + i * ZROWS
            pltpu.sync_copy(acc_sh.at[pl.ds(r0, ZROWS)], rows_v)
            pltpu.sync_copy(rows_v, out_hbm.at[pl.ds(obase + r0, ZROWS)])


ROWS_BLK = 1000
GRID1 = N // ROWS_BLK


def _dense1_body(p0_ref, p1_ref, wg_ref, bg_ref, w1_ref, b1_ref,
                 y_ref, s1_ref, s2_ref):
    agg = p0_ref[...] + p1_ref[...]
    x1 = jnp.dot(agg, wg_ref[...], preferred_element_type=jnp.float32)
    x1 = jnp.maximum(x1 + bg_ref[...], 0.0)
    y = jnp.dot(x1, w1_ref[...], preferred_element_type=jnp.float32) + b1_ref[...]
    y_ref[...] = y

    @pl.when(pl.program_id(0) == 0)
    def _():
        s1_ref[...] = jnp.zeros_like(s1_ref)
        s2_ref[...] = jnp.zeros_like(s2_ref)

    s1_ref[...] += jnp.sum(y, axis=0, keepdims=True)
    s2_ref[...] += jnp.sum(y * y, axis=0, keepdims=True)


def _dense2_body(y_ref, s1_ref, s2_ref, gamma_ref, beta_ref, w2_ref, b2_ref,
                 out_ref):
    mean = s1_ref[...] / N
    var = s2_ref[...] / N - mean * mean
    inv = gamma_ref[...] * lax.rsqrt(var + 1e-5)
    xh = (y_ref[...] - mean) * inv + beta_ref[...]
    z = jnp.maximum(xh, 0.0)
    out_ref[...] = (
        jnp.dot(z, w2_ref[...], preferred_element_type=jnp.float32) + b2_ref[...]
    )


def kernel(edge_index, edge_weight, feat, W_gcn, b_gcn, W1, b1, gamma, beta,
           W2, b2):
    src = edge_index[0].astype(jnp.int32)
    dst = edge_index[1].astype(jnp.int32)

    dst3 = dst.reshape(NW, NCHUNK, CH)
    partials = _build_agg()(src, dst3, edge_weight, feat)
    p0 = partials[:N]
    p1 = partials[N:]

    bg2 = b_gcn.reshape(1, DH)
    b12 = b1.reshape(1, DH)
    g2 = gamma.reshape(1, DH)
    be2 = beta.reshape(1, DH)
    b22 = b2.reshape(1, DO)

    y, s1, s2 = pl.pallas_call(
        _dense1_body,
        grid=(GRID1,),
        in_specs=[
            pl.BlockSpec((ROWS_BLK, DF), lambda i: (i, 0)),
            pl.BlockSpec((ROWS_BLK, DF), lambda i: (i, 0)),
            pl.BlockSpec((DF, DH), lambda i: (0, 0)),
            pl.BlockSpec((1, DH), lambda i: (0, 0)),
            pl.BlockSpec((DH, DH), lambda i: (0, 0)),
            pl.BlockSpec((1, DH), lambda i: (0, 0)),
        ],
        out_specs=[
            pl.BlockSpec((ROWS_BLK, DH), lambda i: (i, 0)),
            pl.BlockSpec((1, DH), lambda i: (0, 0)),
            pl.BlockSpec((1, DH), lambda i: (0, 0)),
        ],
        out_shape=[
            jax.ShapeDtypeStruct((N, DH), jnp.float32),
            jax.ShapeDtypeStruct((1, DH), jnp.float32),
            jax.ShapeDtypeStruct((1, DH), jnp.float32),
        ],
    )(p0, p1, W_gcn, bg2, W1, b12)

    out = pl.pallas_call(
        _dense2_body,
        grid=(GRID1,),
        in_specs=[
            pl.BlockSpec((ROWS_BLK, DH), lambda i: (i, 0)),
            pl.BlockSpec((1, DH), lambda i: (0, 0)),
            pl.BlockSpec((1, DH), lambda i: (0, 0)),
            pl.BlockSpec((1, DH), lambda i: (0, 0)),
            pl.BlockSpec((1, DH), lambda i: (0, 0)),
            pl.BlockSpec((DH, DO), lambda i: (0, 0)),
            pl.BlockSpec((1, DO), lambda i: (0, 0)),
        ],
        out_specs=pl.BlockSpec((ROWS_BLK, DO), lambda i: (i, 0)),
        out_shape=jax.ShapeDtypeStruct((N, DO), jnp.float32),
    )(y, s1, s2, g2, be2, W2, b22)

    return out
